# baseline (speedup 1.0000x reference)
import functools

import jax
import jax.numpy as jnp
from jax import lax
from jax.experimental import pallas as pl
from jax.experimental.pallas import tpu as pltpu
from jax.experimental.pallas import tpu_sc as plsc

BATCH = 16384
LARGE_DIM = 32
MIDDLE_DIM = 48
HIDDEN = 256
OUT_DIM = 128
NUM_MIDDLE = 1000000

NC = 2
NS = 16
NW = NC * NS
BPW = BATCH // NW
WAVE = 256
NWAVE = BPW // WAVE

TBLK = 16384

_sc_mesh = plsc.VectorSubcoreMesh(core_axis_name="c", subcore_axis_name="s")


def _transpose_body(i_ref, eye_ref, o_ref):
    o_ref[...] = lax.dot_general(
        i_ref[...], eye_ref[...], (((0,), (0,)), ((), ())),
        preferred_element_type=jnp.float32)


_transpose_mid = pl.pallas_call(
    _transpose_body,
    grid=(pl.cdiv(NUM_MIDDLE, TBLK),),
    in_specs=[
        pl.BlockSpec((MIDDLE_DIM, TBLK), lambda i: (0, i)),
        pl.BlockSpec((MIDDLE_DIM, MIDDLE_DIM), lambda i: (0, 0)),
    ],
    out_specs=pl.BlockSpec((TBLK, MIDDLE_DIM), lambda i: (i, 0)),
    out_shape=jax.ShapeDtypeStruct((NUM_MIDDLE, MIDDLE_DIM), jnp.float32),
)


@functools.partial(
    pl.kernel,
    out_type=(
        jax.ShapeDtypeStruct((BATCH, LARGE_DIM), jnp.float32),
        jax.ShapeDtypeStruct((BATCH, MIDDLE_DIM), jnp.float32),
    ),
    mesh=_sc_mesh,
    scratch_types=[
        pltpu.VMEM((BPW,), jnp.int32),
        pltpu.VMEM((BPW,), jnp.int32),
        pltpu.VMEM((WAVE, LARGE_DIM), jnp.float32),
        pltpu.VMEM((WAVE, MIDDLE_DIM), jnp.float32),
        pltpu.SemaphoreType.DMA,
    ],
)
def _sc_gather(lids_hbm, mids_hbm, ltab_hbm, mtab_hbm, lout_hbm, mout_hbm,
               lidx_v, midx_v, lrows_v, mrows_v, sem):
    wid = lax.axis_index("s") * NC + lax.axis_index("c")
    base = wid * BPW
    pltpu.sync_copy(lids_hbm.at[pl.ds(base, BPW)], lidx_v)
    pltpu.sync_copy(mids_hbm.at[pl.ds(base, BPW)], midx_v)
    for w in range(NWAVE):
        off = w * WAVE

        def _fire(g, carry):
            vl = lidx_v[pl.ds(off + g * 16, 16)]
            vm = midx_v[pl.ds(off + g * 16, 16)]
            for k in range(16):
                i = g * 16 + k
                pltpu.async_copy(ltab_hbm.at[pl.ds(vl[k], 1)],
                                 lrows_v.at[pl.ds(i, 1)], sem)
                pltpu.async_copy(mtab_hbm.at[pl.ds(vm[k], 1)],
                                 mrows_v.at[pl.ds(i, 1)], sem)
            return carry

        lax.fori_loop(0, WAVE // 16, _fire, 0)
        pltpu.make_async_copy(
            lout_hbm.at[pl.ds(base + off, WAVE)], lrows_v, sem).wait()
        pltpu.make_async_copy(
            mout_hbm.at[pl.ds(base + off, WAVE)], mrows_v, sem).wait()
        pltpu.sync_copy(lrows_v, lout_hbm.at[pl.ds(base + off, WAVE)])
        pltpu.sync_copy(mrows_v, mout_hbm.at[pl.ds(base + off, WAVE)])


BM = 2048


def _mlp_body(l_ref, m_ref, w1l_ref, w1m_ref, b1_ref, w2_ref, b2_ref, o_ref):
    h = jnp.dot(l_ref[...], w1l_ref[...], preferred_element_type=jnp.float32)
    h = h + jnp.dot(m_ref[...], w1m_ref[...],
                    preferred_element_type=jnp.float32)
    h = jnp.maximum(h + b1_ref[...], 0.0)
    o = jnp.dot(h, w2_ref[...], preferred_element_type=jnp.float32)
    o_ref[...] = jnp.maximum(o + b2_ref[...], 0.0)


_mlp = pl.pallas_call(
    _mlp_body,
    grid=(BATCH // BM,),
    in_specs=[
        pl.BlockSpec((BM, LARGE_DIM), lambda i: (i, 0)),
        pl.BlockSpec((BM, MIDDLE_DIM), lambda i: (i, 0)),
        pl.BlockSpec((LARGE_DIM, HIDDEN), lambda i: (0, 0)),
        pl.BlockSpec((MIDDLE_DIM, HIDDEN), lambda i: (0, 0)),
        pl.BlockSpec((1, HIDDEN), lambda i: (0, 0)),
        pl.BlockSpec((HIDDEN, OUT_DIM), lambda i: (0, 0)),
        pl.BlockSpec((1, OUT_DIM), lambda i: (0, 0)),
    ],
    out_specs=pl.BlockSpec((BM, OUT_DIM), lambda i: (i, 0)),
    out_shape=jax.ShapeDtypeStruct((BATCH, OUT_DIM), jnp.float32),
)


def kernel(large_category_ids, middle_category_ids, large_table, middle_table,
           W1, b1, W2, b2):
    lids = large_category_ids.astype(jnp.int32)
    mids = middle_category_ids.astype(jnp.int32)
    mtab_rm = _transpose_mid(middle_table.T,
                             jnp.eye(MIDDLE_DIM, dtype=jnp.float32))
    l_emb, m_emb = _sc_gather(lids, mids, large_table, mtab_rm)
    return _mlp(l_emb, m_emb, W1[:LARGE_DIM], W1[LARGE_DIM:],
                b1.reshape(1, HIDDEN), W2, b2.reshape(1, OUT_DIM))


# speedup vs baseline: 1.0194x; 1.0194x over previous
import functools

import jax
import jax.numpy as jnp
from jax import lax
from jax.experimental import pallas as pl
from jax.experimental.pallas import tpu as pltpu
from jax.experimental.pallas import tpu_sc as plsc

BATCH = 16384
LARGE_DIM = 32
MIDDLE_DIM = 48
HIDDEN = 256
OUT_DIM = 128
NUM_MIDDLE = 1000000

NC = 2
NS = 16
NW = NC * NS
BPW = BATCH // NW
WAVE = 256
NWAVE = BPW // WAVE

TBLK = 32768

_sc_mesh = plsc.VectorSubcoreMesh(core_axis_name="c", subcore_axis_name="s")


def _transpose_body(i_ref, eye_ref, o_ref):
    o_ref[...] = lax.dot_general(
        i_ref[...], eye_ref[...], (((0,), (0,)), ((), ())),
        preferred_element_type=jnp.float32)


_transpose_mid = pl.pallas_call(
    _transpose_body,
    grid=(pl.cdiv(NUM_MIDDLE, TBLK),),
    in_specs=[
        pl.BlockSpec((MIDDLE_DIM, TBLK), lambda i: (0, i)),
        pl.BlockSpec((MIDDLE_DIM, MIDDLE_DIM), lambda i: (0, 0)),
    ],
    out_specs=pl.BlockSpec((TBLK, MIDDLE_DIM), lambda i: (i, 0)),
    out_shape=jax.ShapeDtypeStruct((NUM_MIDDLE, MIDDLE_DIM), jnp.float32),
)


@functools.partial(
    pl.kernel,
    out_type=(
        jax.ShapeDtypeStruct((BATCH, LARGE_DIM), jnp.float32),
        jax.ShapeDtypeStruct((BATCH, MIDDLE_DIM), jnp.float32),
    ),
    mesh=_sc_mesh,
    scratch_types=[
        pltpu.VMEM((BPW,), jnp.int32),
        pltpu.VMEM((BPW,), jnp.int32),
        pltpu.VMEM((WAVE, LARGE_DIM), jnp.float32),
        pltpu.VMEM((WAVE, MIDDLE_DIM), jnp.float32),
        pltpu.SemaphoreType.DMA,
    ],
)
def _sc_gather(lids_hbm, mids_hbm, ltab_hbm, mtab_hbm, lout_hbm, mout_hbm,
               lidx_v, midx_v, lrows_v, mrows_v, sem):
    wid = lax.axis_index("s") * NC + lax.axis_index("c")
    base = wid * BPW
    pltpu.sync_copy(lids_hbm.at[pl.ds(base, BPW)], lidx_v)
    pltpu.sync_copy(mids_hbm.at[pl.ds(base, BPW)], midx_v)
    for w in range(NWAVE):
        off = w * WAVE

        def _fire(g, carry):
            vl = lidx_v[pl.ds(off + g * 16, 16)]
            vm = midx_v[pl.ds(off + g * 16, 16)]
            for k in range(16):
                i = g * 16 + k
                pltpu.async_copy(ltab_hbm.at[pl.ds(vl[k], 1)],
                                 lrows_v.at[pl.ds(i, 1)], sem)
                pltpu.async_copy(mtab_hbm.at[pl.ds(vm[k], 1)],
                                 mrows_v.at[pl.ds(i, 1)], sem)
            return carry

        lax.fori_loop(0, WAVE // 16, _fire, 0)
        pltpu.make_async_copy(
            lout_hbm.at[pl.ds(base + off, WAVE)], lrows_v, sem).wait()
        pltpu.make_async_copy(
            mout_hbm.at[pl.ds(base + off, WAVE)], mrows_v, sem).wait()
        pltpu.sync_copy(lrows_v, lout_hbm.at[pl.ds(base + off, WAVE)])
        pltpu.sync_copy(mrows_v, mout_hbm.at[pl.ds(base + off, WAVE)])


BM = 2048


def _mlp_body(l_ref, m_ref, w1l_ref, w1m_ref, b1_ref, w2_ref, b2_ref, o_ref):
    h = jnp.dot(l_ref[...], w1l_ref[...], preferred_element_type=jnp.float32)
    h = h + jnp.dot(m_ref[...], w1m_ref[...],
                    preferred_element_type=jnp.float32)
    h = jnp.maximum(h + b1_ref[...], 0.0)
    o = jnp.dot(h, w2_ref[...], preferred_element_type=jnp.float32)
    o_ref[...] = jnp.maximum(o + b2_ref[...], 0.0)


_mlp = pl.pallas_call(
    _mlp_body,
    grid=(BATCH // BM,),
    in_specs=[
        pl.BlockSpec((BM, LARGE_DIM), lambda i: (i, 0)),
        pl.BlockSpec((BM, MIDDLE_DIM), lambda i: (i, 0)),
        pl.BlockSpec((LARGE_DIM, HIDDEN), lambda i: (0, 0)),
        pl.BlockSpec((MIDDLE_DIM, HIDDEN), lambda i: (0, 0)),
        pl.BlockSpec((1, HIDDEN), lambda i: (0, 0)),
        pl.BlockSpec((HIDDEN, OUT_DIM), lambda i: (0, 0)),
        pl.BlockSpec((1, OUT_DIM), lambda i: (0, 0)),
    ],
    out_specs=pl.BlockSpec((BM, OUT_DIM), lambda i: (i, 0)),
    out_shape=jax.ShapeDtypeStruct((BATCH, OUT_DIM), jnp.float32),
)


def kernel(large_category_ids, middle_category_ids, large_table, middle_table,
           W1, b1, W2, b2):
    lids = large_category_ids.astype(jnp.int32)
    mids = middle_category_ids.astype(jnp.int32)
    mtab_rm = _transpose_mid(middle_table.T,
                             jnp.eye(MIDDLE_DIM, dtype=jnp.float32))
    l_emb, m_emb = _sc_gather(lids, mids, large_table, mtab_rm)
    return _mlp(l_emb, m_emb, W1[:LARGE_DIM], W1[LARGE_DIM:],
                b1.reshape(1, HIDDEN), W2, b2.reshape(1, OUT_DIM))


# speedup vs baseline: 1.0202x; 1.0008x over previous
import functools

import jax
import jax.numpy as jnp
from jax import lax
from jax.experimental import pallas as pl
from jax.experimental.pallas import tpu as pltpu
from jax.experimental.pallas import tpu_sc as plsc

BATCH = 16384
LARGE_DIM = 32
MIDDLE_DIM = 48
HIDDEN = 256
OUT_DIM = 128
NUM_MIDDLE = 1000000

NC = 2
NS = 16
NW = NC * NS
BPW = BATCH // NW
WAVE = 256
NWAVE = BPW // WAVE

TBLK = 32768

_sc_mesh = plsc.VectorSubcoreMesh(core_axis_name="c", subcore_axis_name="s")


def _transpose_body(i_ref, eye_ref, o_ref):
    o_ref[:, : MIDDLE_DIM] = lax.dot_general(
        i_ref[...], eye_ref[...], (((0,), (0,)), ((), ())),
        preferred_element_type=jnp.float32)


_transpose_mid = pl.pallas_call(
    _transpose_body,
    grid=(pl.cdiv(NUM_MIDDLE, TBLK),),
    in_specs=[
        pl.BlockSpec((MIDDLE_DIM, TBLK), lambda i: (0, i)),
        pl.BlockSpec((MIDDLE_DIM, MIDDLE_DIM), lambda i: (0, 0)),
    ],
    out_specs=pl.BlockSpec((TBLK, 128), lambda i: (i, 0)),
    out_shape=jax.ShapeDtypeStruct((NUM_MIDDLE, 128), jnp.float32),
)


@functools.partial(
    pl.kernel,
    out_type=(
        jax.ShapeDtypeStruct((BATCH, LARGE_DIM), jnp.float32),
        jax.ShapeDtypeStruct((BATCH, 128), jnp.float32),
    ),
    mesh=_sc_mesh,
    scratch_types=[
        pltpu.VMEM((BPW,), jnp.int32),
        pltpu.VMEM((BPW,), jnp.int32),
        pltpu.VMEM((WAVE, LARGE_DIM), jnp.float32),
        pltpu.VMEM((WAVE, 128), jnp.float32),
        pltpu.SemaphoreType.DMA,
    ],
)
def _sc_gather(lids_hbm, mids_hbm, ltab_hbm, mtab_hbm, lout_hbm, mout_hbm,
               lidx_v, midx_v, lrows_v, mrows_v, sem):
    wid = lax.axis_index("s") * NC + lax.axis_index("c")
    base = wid * BPW
    pltpu.sync_copy(lids_hbm.at[pl.ds(base, BPW)], lidx_v)
    pltpu.sync_copy(mids_hbm.at[pl.ds(base, BPW)], midx_v)
    for w in range(NWAVE):
        off = w * WAVE

        def _fire(g, carry):
            vl = lidx_v[pl.ds(off + g * 16, 16)]
            vm = midx_v[pl.ds(off + g * 16, 16)]
            for k in range(16):
                i = g * 16 + k
                pltpu.async_copy(ltab_hbm.at[pl.ds(vl[k], 1)],
                                 lrows_v.at[pl.ds(i, 1)], sem)
                pltpu.async_copy(mtab_hbm.at[pl.ds(vm[k], 1)],
                                 mrows_v.at[pl.ds(i, 1)], sem)
            return carry

        lax.fori_loop(0, WAVE // 16, _fire, 0)
        pltpu.make_async_copy(
            lout_hbm.at[pl.ds(base + off, WAVE)], lrows_v, sem).wait()
        pltpu.make_async_copy(
            mout_hbm.at[pl.ds(base + off, WAVE)], mrows_v, sem).wait()
        pltpu.sync_copy(lrows_v, lout_hbm.at[pl.ds(base + off, WAVE)])
        pltpu.sync_copy(mrows_v, mout_hbm.at[pl.ds(base + off, WAVE)])


BM = 2048


def _mlp_body(l_ref, m_ref, w1l_ref, w1m_ref, b1_ref, w2_ref, b2_ref, o_ref):
    h = jnp.dot(l_ref[...], w1l_ref[...], preferred_element_type=jnp.float32)
    h = h + jnp.dot(m_ref[...][:, : MIDDLE_DIM], w1m_ref[...],
                    preferred_element_type=jnp.float32)
    h = jnp.maximum(h + b1_ref[...], 0.0)
    o = jnp.dot(h, w2_ref[...], preferred_element_type=jnp.float32)
    o_ref[...] = jnp.maximum(o + b2_ref[...], 0.0)


_mlp = pl.pallas_call(
    _mlp_body,
    grid=(BATCH // BM,),
    in_specs=[
        pl.BlockSpec((BM, LARGE_DIM), lambda i: (i, 0)),
        pl.BlockSpec((BM, 128), lambda i: (i, 0)),
        pl.BlockSpec((LARGE_DIM, HIDDEN), lambda i: (0, 0)),
        pl.BlockSpec((MIDDLE_DIM, HIDDEN), lambda i: (0, 0)),
        pl.BlockSpec((1, HIDDEN), lambda i: (0, 0)),
        pl.BlockSpec((HIDDEN, OUT_DIM), lambda i: (0, 0)),
        pl.BlockSpec((1, OUT_DIM), lambda i: (0, 0)),
    ],
    out_specs=pl.BlockSpec((BM, OUT_DIM), lambda i: (i, 0)),
    out_shape=jax.ShapeDtypeStruct((BATCH, OUT_DIM), jnp.float32),
)


def kernel(large_category_ids, middle_category_ids, large_table, middle_table,
           W1, b1, W2, b2):
    lids = large_category_ids.astype(jnp.int32)
    mids = middle_category_ids.astype(jnp.int32)
    mtab_rm = _transpose_mid(middle_table.T,
                             jnp.eye(MIDDLE_DIM, dtype=jnp.float32))
    l_emb, m_emb = _sc_gather(lids, mids, large_table, mtab_rm)
    return _mlp(l_emb, m_emb, W1[:LARGE_DIM], W1[LARGE_DIM:],
                b1.reshape(1, HIDDEN), W2, b2.reshape(1, OUT_DIM))
